# Initial kernel scaffold; baseline (speedup 1.0000x reference)
import jax
import jax.numpy as jnp
from jax.experimental import pallas as pl


def kernel(x, edge_index, edge_attr, W0, W1, W2, W3, fcw0, fcb0, fcw1, fcb1, fcw2, fcb2, fcw3, fcb3):
    raise NotImplementedError("write your pallas kernel here")


# speedup vs baseline: 1.3945x; 1.3945x over previous
import functools

import jax
import jax.numpy as jnp
from jax import lax
from jax.experimental import pallas as pl

N = 10000
E = 320000
NBASIS = 8
NPAIR = 7
NPAD = 10240
NCORES = 2
NSUB = 16
NTILES = NCORES * NSUB
C = 128
EPT = 10240
EP = NTILES * EPT
NCHUNK = EPT // C
ZR = 128
NBLK = 400
SCALE = 1.0 / 128.0
INV_H = 3.5


def _prep_body(dst_ref, src_ref, attr_ref, g0_ref, f_ref, dste_ref):
    dst = dst_ref[...]
    src = src_ref[...]
    t = jnp.clip(attr_ref[...], -1.0, 1.0)
    u = (t + 1.0) * INV_H
    a0 = jnp.minimum(jnp.floor(u), 6.0)
    f_ref[...] = u - a0
    mask = dst != src
    lane = lax.broadcasted_iota(jnp.int32, dst.shape, 1)
    g0_ref[...] = jnp.where(mask, src * NPAIR + a0.astype(jnp.int32),
                            lane * NPAIR)
    dste_ref[...] = jnp.where(mask, dst, N + lane)


def _prep_edges(dst, src, attr):
    r = EP // 128
    out = pl.pallas_call(
        _prep_body,
        out_shape=[
            jax.ShapeDtypeStruct((r, 128), jnp.int32),
            jax.ShapeDtypeStruct((r, 128), jnp.float32),
            jax.ShapeDtypeStruct((r, 128), jnp.int32),
        ],
    )(dst.reshape(r, 128), src.reshape(r, 128), attr.reshape(r, 128))
    return out[0], out[1], out[2]


def _tc0_body(x_ref, wc_ref, fw_ref, fb_ref, yc_ref, s_ref):
    a = x_ref[...]
    yc_ref[...] = jnp.dot(a, wc_ref[...], preferred_element_type=jnp.float32)
    s_ref[...] = jnp.dot(a, fw_ref[...], preferred_element_type=jnp.float32) + fb_ref[...]


def _tc0(x, wc, fw, fb):
    d = x.shape[1]
    oc = wc.shape[1]
    o = fw.shape[1]
    return pl.pallas_call(
        _tc0_body,
        grid=(N // NBLK,),
        in_specs=[
            pl.BlockSpec((NBLK, d), lambda i: (i, 0)),
            pl.BlockSpec((d, oc), lambda i: (0, 0)),
            pl.BlockSpec((d, o), lambda i: (0, 0)),
            pl.BlockSpec((1, o), lambda i: (0, 0)),
        ],
        out_specs=[
            pl.BlockSpec((NBLK, oc), lambda i: (i, 0)),
            pl.BlockSpec((NBLK, o), lambda i: (i, 0)),
        ],
        out_shape=[
            jax.ShapeDtypeStruct((N, oc), jnp.float32),
            jax.ShapeDtypeStruct((N, o), jnp.float32),
        ],
    )(x, wc, fw, fb.reshape(1, o))


def _tcmid_body(aa_ref, ab_ref, sp_ref, wc_ref, fw_ref, fb_ref, yc_ref, sn_ref,
                *, concat, res):
    conv = aa_ref[0] + ab_ref[0]
    if concat:
        ans = jnp.concatenate([conv, sp_ref[...]], axis=1)
    else:
        ans = conv + sp_ref[...]
    a = jnp.maximum(ans, 0.0)
    yc_ref[...] = jnp.dot(a, wc_ref[...], preferred_element_type=jnp.float32)
    sn = jnp.dot(a, fw_ref[...], preferred_element_type=jnp.float32) + fb_ref[...]
    if res:
        sn = sn + ans
    sn_ref[...] = sn


def _tcmid(acc, sp, wc, fw, fb, concat, res):
    op = acc.shape[2]
    d = wc.shape[0]
    oc = wc.shape[1]
    o = fw.shape[1]
    so = d if res else o
    body = functools.partial(_tcmid_body, concat=concat, res=res)
    return pl.pallas_call(
        body,
        grid=(N // NBLK,),
        in_specs=[
            pl.BlockSpec((1, NBLK, op), lambda i: (0, i, 0)),
            pl.BlockSpec((1, NBLK, op), lambda i: (1, i, 0)),
            pl.BlockSpec((NBLK, sp.shape[1]), lambda i: (i, 0)),
            pl.BlockSpec((d, oc), lambda i: (0, 0)),
            pl.BlockSpec((d, o), lambda i: (0, 0)),
            pl.BlockSpec((1, o), lambda i: (0, 0)),
        ],
        out_specs=[
            pl.BlockSpec((NBLK, oc), lambda i: (i, 0)),
            pl.BlockSpec((NBLK, so), lambda i: (i, 0)),
        ],
        out_shape=[
            jax.ShapeDtypeStruct((N, oc), jnp.float32),
            jax.ShapeDtypeStruct((N, so), jnp.float32),
        ],
    )(acc, acc, sp, wc, fw, fb.reshape(1, o))


def _tcf_body(aa_ref, ab_ref, sp_ref, o_ref):
    conv = aa_ref[0] + ab_ref[0]
    o_ref[...] = (conv[:, :2] + sp_ref[...]) * SCALE


def _tcf(acc, sp):
    op = acc.shape[2]
    return pl.pallas_call(
        _tcf_body,
        grid=(N // NBLK,),
        in_specs=[
            pl.BlockSpec((1, NBLK, op), lambda i: (0, i, 0)),
            pl.BlockSpec((1, NBLK, op), lambda i: (1, i, 0)),
            pl.BlockSpec((NBLK, 2), lambda i: (i, 0)),
        ],
        out_specs=pl.BlockSpec((NBLK, 2), lambda i: (i, 0)),
        out_shape=jax.ShapeDtypeStruct((N, 2), jnp.float32),
    )(acc, acc, sp)


def _edge_stage(yc, g0f, ff, dstf, o):
    rows = yc.reshape(N * NPAIR, 128)[g0f]
    msg = rows[:, :o] + ff[:, None] * (rows[:, o:2 * o] - rows[:, :o])
    acc = jnp.zeros((NPAD, o), jnp.float32).at[dstf].add(msg)
    return jnp.stack([acc, jnp.zeros_like(acc)])


def _wpair(w):
    nb, d, o = w.shape
    out = jnp.zeros((d, NPAIR * 128), w.dtype)
    for a in range(NPAIR):
        out = out.at[:, a * 128:a * 128 + o].set(w[a])
        out = out.at[:, a * 128 + o:a * 128 + 2 * o].set(w[a + 1])
    return out


def kernel(x, edge_index, edge_attr, W0, W1, W2, W3,
           fcw0, fcb0, fcw1, fcb1, fcw2, fcb2, fcw3, fcb3):
    dst = jnp.pad(edge_index[0], (0, EP - E))
    src = jnp.pad(edge_index[1], (0, EP - E))
    attr = jnp.pad(edge_attr[:, 0], (0, EP - E))
    wc0 = _wpair(W0)
    wc1 = _wpair(W1)
    wc2 = _wpair(W2)
    w3p = jnp.zeros((NBASIS, W3.shape[1], 16), jnp.float32).at[:, :, :2].set(W3)
    wc3 = _wpair(w3p)

    g0, f, dste = _prep_edges(dst, src, attr)
    g0f = g0.reshape(EP)
    ff = f.reshape(EP)
    dstf = dste.reshape(EP)

    yc0, s0 = _tc0(x, wc0, fcw0, fcb0)
    acc0 = _edge_stage(yc0, g0f, ff, dstf, 32)

    yc1, s1 = _tcmid(acc0, s0, wc1, fcw1, fcb1, concat=True, res=True)
    acc1 = _edge_stage(yc1, g0f, ff, dstf, 64)

    yc2, s2 = _tcmid(acc1, s1, wc2, fcw2, fcb2, concat=False, res=True)
    acc2 = _edge_stage(yc2, g0f, ff, dstf, 64)

    yc3, s3 = _tcmid(acc2, s2, wc3, fcw3, fcb3, concat=False, res=False)
    acc3 = _edge_stage(yc3, g0f, ff, dstf, 16)

    return _tcf(acc3, s3)
